# Initial kernel scaffold; baseline (speedup 1.0000x reference)
import jax
import jax.numpy as jnp
from jax.experimental import pallas as pl


def kernel(x, w1, b1, g1, bt1, w2, b2, g2, bt2):
    raise NotImplementedError("write your pallas kernel here")


# speedup vs baseline: 1.1871x; 1.1871x over previous
import functools

import jax
import jax.numpy as jnp
from jax.experimental import pallas as pl
from jax.experimental.pallas import tpu as pltpu

_EPS = 1e-5
_VMEM_LIMIT = 64 * 1024 * 1024


def _pick_tile(n: int) -> int:
    t = min(n, 2048)
    while t > 8 and n % t:
        t //= 2
    return t


def _stage1_kernel(x_ref, w1t_ref, b1_ref, h1_ref, sum_ref, ssq_ref):
    i = pl.program_id(0)

    @pl.when(i == 0)
    def _():
        sum_ref[...] = jnp.zeros_like(sum_ref)
        ssq_ref[...] = jnp.zeros_like(ssq_ref)

    xb = x_ref[...].astype(jnp.bfloat16)
    h = jax.lax.dot_general(
        xb, w1t_ref[...], (((1,), (1,)), ((), ())),
        preferred_element_type=jnp.float32) + b1_ref[...]
    h1_ref[...] = h.astype(jnp.bfloat16)
    sum_ref[...] += jnp.sum(h, axis=0, keepdims=True)
    ssq_ref[...] += jnp.sum(h * h, axis=0, keepdims=True)


def _stage2_kernel(h1_ref, sum1_ref, ssq1_ref, g1_ref, bt1_ref,
                   w2t_ref, b2_ref, h2_ref, sum_ref, ssq_ref, *, n_total):
    i = pl.program_id(0)

    @pl.when(i == 0)
    def _():
        sum_ref[...] = jnp.zeros_like(sum_ref)
        ssq_ref[...] = jnp.zeros_like(ssq_ref)

    mu = sum1_ref[...] * (1.0 / n_total)
    var = jnp.maximum(ssq1_ref[...] * (1.0 / n_total) - mu * mu, 0.0)
    s = g1_ref[...] * jax.lax.rsqrt(var + _EPS)
    c = bt1_ref[...] - s * mu

    a = jnp.maximum(h1_ref[...].astype(jnp.float32) * s + c, 0.0)
    h = jax.lax.dot_general(
        a.astype(jnp.bfloat16), w2t_ref[...], (((1,), (1,)), ((), ())),
        preferred_element_type=jnp.float32) + b2_ref[...]
    h2_ref[...] = h.astype(jnp.bfloat16)
    sum_ref[...] += jnp.sum(h, axis=0, keepdims=True)
    ssq_ref[...] += jnp.sum(h * h, axis=0, keepdims=True)


def _stage3_kernel(h2_ref, sum2_ref, ssq2_ref, g2_ref, bt2_ref, o_ref,
                   *, n_total):
    mu = sum2_ref[...] * (1.0 / n_total)
    var = jnp.maximum(ssq2_ref[...] * (1.0 / n_total) - mu * mu, 0.0)
    s = g2_ref[...] * jax.lax.rsqrt(var + _EPS)
    c = bt2_ref[...] - s * mu
    o_ref[...] = jnp.maximum(h2_ref[...].astype(jnp.float32) * s + c, 0.0)


def kernel(x, w1, b1, g1, bt1, w2, b2, g2, bt2):
    B, C, H, W = x.shape
    N = B * H * W
    Fi = w1.shape[0]
    Fo = w2.shape[0]

    xn = jnp.reshape(x, (N, C))
    w1b = w1.astype(jnp.bfloat16)
    w2b = w2.astype(jnp.bfloat16)
    b1r = b1.reshape(1, Fi)
    g1r = g1.reshape(1, Fi)
    bt1r = bt1.reshape(1, Fi)
    b2r = b2.reshape(1, Fo)
    g2r = g2.reshape(1, Fo)
    bt2r = bt2.reshape(1, Fo)

    tile = _pick_tile(N)
    grid = (N // tile,)

    def row_spec(cols):
        return pl.BlockSpec((tile, cols), lambda i: (i, 0))

    def const_spec(a):
        return pl.BlockSpec(a.shape, lambda i: (0, 0))

    def stat_out(f):
        return (pl.BlockSpec((1, f), lambda i: (0, 0)),
                pl.BlockSpec((1, f), lambda i: (0, 0)))

    reduce_cp = pltpu.CompilerParams(
        dimension_semantics=("arbitrary",), vmem_limit_bytes=_VMEM_LIMIT)
    parallel_cp = pltpu.CompilerParams(
        dimension_semantics=("parallel",), vmem_limit_bytes=_VMEM_LIMIT)

    h1, sum1, ssq1 = pl.pallas_call(
        _stage1_kernel,
        grid=grid,
        in_specs=[row_spec(C), const_spec(w1b), const_spec(b1r)],
        out_specs=(row_spec(Fi),) + stat_out(Fi),
        out_shape=(jax.ShapeDtypeStruct((N, Fi), jnp.bfloat16),
                   jax.ShapeDtypeStruct((1, Fi), jnp.float32),
                   jax.ShapeDtypeStruct((1, Fi), jnp.float32)),
        compiler_params=reduce_cp,
    )(xn, w1b, b1r)

    h2, sum2, ssq2 = pl.pallas_call(
        functools.partial(_stage2_kernel, n_total=N),
        grid=grid,
        in_specs=[row_spec(Fi), const_spec(sum1), const_spec(ssq1),
                  const_spec(g1r), const_spec(bt1r),
                  const_spec(w2b), const_spec(b2r)],
        out_specs=(row_spec(Fo),) + stat_out(Fo),
        out_shape=(jax.ShapeDtypeStruct((N, Fo), jnp.bfloat16),
                   jax.ShapeDtypeStruct((1, Fo), jnp.float32),
                   jax.ShapeDtypeStruct((1, Fo), jnp.float32)),
        compiler_params=reduce_cp,
    )(h1, sum1, ssq1, g1r, bt1r, w2b, b2r)

    out_n = pl.pallas_call(
        functools.partial(_stage3_kernel, n_total=N),
        grid=grid,
        in_specs=[row_spec(Fo), const_spec(sum2), const_spec(ssq2),
                  const_spec(g2r), const_spec(bt2r)],
        out_specs=row_spec(Fo),
        out_shape=jax.ShapeDtypeStruct((N, Fo), jnp.float32),
        compiler_params=parallel_cp,
    )(h2, sum2, ssq2, g2r, bt2r)

    return jnp.reshape(out_n, (B, Fo, H, W))


# speedup vs baseline: 1.2475x; 1.0509x over previous
import functools

import jax
import jax.numpy as jnp
from jax.experimental import pallas as pl
from jax.experimental.pallas import tpu as pltpu

_EPS = 1e-5
_VMEM_LIMIT = 60 * 1024 * 1024


def _pick_tile(n: int) -> int:
    t = min(n, 2048)
    while t > 8 and n % t:
        t //= 2
    return t


def _fused_kernel(x_ref, w1_ref, b1_ref, g1_ref, bt1_ref,
                  w2_ref, b2_ref, g2_ref, bt2_ref, o_ref,
                  h1_ref, h2_ref, sum1_ref, ssq1_ref, sum2_ref, ssq2_ref,
                  *, n_total, tile):
    p = pl.program_id(0)
    i = pl.program_id(1)
    inv_n = 1.0 / n_total

    @pl.when((p == 0) & (i == 0))
    def _():
        sum1_ref[...] = jnp.zeros_like(sum1_ref)
        ssq1_ref[...] = jnp.zeros_like(ssq1_ref)
        sum2_ref[...] = jnp.zeros_like(sum2_ref)
        ssq2_ref[...] = jnp.zeros_like(ssq2_ref)

    @pl.when(p == 0)
    def _():
        xb = x_ref[...].astype(jnp.bfloat16)
        h = jax.lax.dot_general(
            xb, w1_ref[...], (((1,), (1,)), ((), ())),
            preferred_element_type=jnp.float32) + b1_ref[...]
        h1_ref[pl.ds(i * tile, tile), :] = h.astype(jnp.bfloat16)
        sum1_ref[...] += jnp.sum(h, axis=0, keepdims=True)
        ssq1_ref[...] += jnp.sum(h * h, axis=0, keepdims=True)

    @pl.when(p == 1)
    def _():
        mu = sum1_ref[...] * inv_n
        var = jnp.maximum(ssq1_ref[...] * inv_n - mu * mu, 0.0)
        s = g1_ref[...] * jax.lax.rsqrt(var + _EPS)
        c = bt1_ref[...] - s * mu
        a = jnp.maximum(h1_ref[pl.ds(i * tile, tile), :].astype(jnp.float32)
                        * s + c, 0.0)
        h = jax.lax.dot_general(
            a.astype(jnp.bfloat16), w2_ref[...], (((1,), (1,)), ((), ())),
            preferred_element_type=jnp.float32) + b2_ref[...]
        h2_ref[pl.ds(i * tile, tile), :] = h.astype(jnp.bfloat16)
        sum2_ref[...] += jnp.sum(h, axis=0, keepdims=True)
        ssq2_ref[...] += jnp.sum(h * h, axis=0, keepdims=True)

    @pl.when(p == 2)
    def _():
        mu = sum2_ref[...] * inv_n
        var = jnp.maximum(ssq2_ref[...] * inv_n - mu * mu, 0.0)
        s = g2_ref[...] * jax.lax.rsqrt(var + _EPS)
        c = bt2_ref[...] - s * mu
        o_ref[...] = jnp.maximum(
            h2_ref[pl.ds(i * tile, tile), :].astype(jnp.float32) * s + c, 0.0)


def kernel(x, w1, b1, g1, bt1, w2, b2, g2, bt2):
    B, C, H, W = x.shape
    N = B * H * W
    Fi = w1.shape[0]
    Fo = w2.shape[0]

    xn = jnp.reshape(x, (N, C))
    w1b = w1.astype(jnp.bfloat16)
    w2b = w2.astype(jnp.bfloat16)
    b1r = b1.reshape(1, Fi)
    g1r = g1.reshape(1, Fi)
    bt1r = bt1.reshape(1, Fi)
    b2r = b2.reshape(1, Fo)
    g2r = g2.reshape(1, Fo)
    bt2r = bt2.reshape(1, Fo)

    tile = _pick_tile(N)
    nsteps = N // tile
    grid = (3, nsteps)
    last = nsteps - 1

    def const_spec(a):
        return pl.BlockSpec(a.shape, lambda p, i: (0, 0))

    x_spec = pl.BlockSpec((tile, C),
                          lambda p, i: (jnp.where(p == 0, i, last), 0))
    o_spec = pl.BlockSpec((tile, Fo),
                          lambda p, i: (jnp.where(p == 2, i, 0), 0))

    out_n = pl.pallas_call(
        functools.partial(_fused_kernel, n_total=N, tile=tile),
        grid=grid,
        in_specs=[x_spec, const_spec(w1b), const_spec(b1r), const_spec(g1r),
                  const_spec(bt1r), const_spec(w2b), const_spec(b2r),
                  const_spec(g2r), const_spec(bt2r)],
        out_specs=o_spec,
        out_shape=jax.ShapeDtypeStruct((N, Fo), jnp.float32),
        scratch_shapes=[
            pltpu.VMEM((N, Fi), jnp.bfloat16),
            pltpu.VMEM((N, Fo), jnp.bfloat16),
            pltpu.VMEM((1, Fi), jnp.float32),
            pltpu.VMEM((1, Fi), jnp.float32),
            pltpu.VMEM((1, Fo), jnp.float32),
            pltpu.VMEM((1, Fo), jnp.float32),
        ],
        compiler_params=pltpu.CompilerParams(
            dimension_semantics=("arbitrary", "arbitrary"),
            vmem_limit_bytes=_VMEM_LIMIT),
    )(xn, w1b, b1r, g1r, bt1r, w2b, b2r, g2r, bt2r)

    return jnp.reshape(out_n, (B, Fo, H, W))
